# baseline (speedup 1.0000x reference)
import functools

import jax
import jax.numpy as jnp
from jax import lax
from jax.experimental import pallas as pl
from jax.experimental.pallas import tpu as pltpu, tpu_sc as plsc

N_NODES = 10000
N_PAD = 10240
GROUP = 128
GROUP_S = 128
NC, NS = 2, 16
NW = NC * NS
ROWS_PER_TILE = N_PAD // NS


def _sc_mesh():
    return plsc.VectorSubcoreMesh(
        core_axis_name="c", subcore_axis_name="s", num_cores=NC, num_subcores=NS
    )


def _k1_body(num_groups, x_hbm, idx_hbm, s_out, hist_out,
             s_sh, idx_v, rows_v, hist_v, sem_i, sem_r):
    c = lax.axis_index("c")
    sid = lax.axis_index("s")
    wid = sid * NC + c

    @pl.loop(0, GROUP_S)
    def _(i):
        @pl.loop(0, 8)
        def _(j):
            rows_v[0, i, pl.ds(j * 16, 16)] = jnp.zeros((16,), jnp.float32)

    @pl.loop(0, ROWS_PER_TILE // GROUP_S)
    def _(r):
        base = sid * ROWS_PER_TILE + r * GROUP_S
        pltpu.sync_copy(rows_v.at[0], s_sh.at[pl.ds(base, GROUP_S)])

    @pl.loop(0, N_PAD // 16)
    def _(i):
        hist_v[pl.ds(i * 16, 16)] = jnp.zeros((16,), jnp.float32)
    plsc.subcore_barrier()

    ones16 = jnp.ones((16,), jnp.float32)
    n_i = (num_groups - wid + NW - 1) // NW

    def start(i, slot):
        g = wid + i * NW
        pltpu.async_copy(idx_hbm.at[pl.ds(g * GROUP_S, GROUP_S)],
                         idx_v.at[slot], sem_i.at[slot])
        pltpu.async_copy(x_hbm.at[pl.ds(g * GROUP_S, GROUP_S)],
                         rows_v.at[slot], sem_r.at[slot])

    start(0, 0)

    @pl.loop(0, n_i)
    def _(i):
        b = lax.rem(i, 2)
        g = wid + i * NW
        pltpu.make_async_copy(idx_hbm.at[pl.ds(g * GROUP_S, GROUP_S)],
                              idx_v.at[b], sem_i.at[b]).wait()
        pltpu.make_async_copy(x_hbm.at[pl.ds(g * GROUP_S, GROUP_S)],
                              rows_v.at[b], sem_r.at[b]).wait()

        @pl.when(i + 1 < n_i)
        def _():
            start(i + 1, 1 - b)

        pltpu.sync_copy(rows_v.at[b], s_sh.at[idx_v.at[b]], add=True)
        for j in range(GROUP_S // 16):
            idx16 = idx_v[b, pl.ds(j * 16, 16)]
            plsc.addupdate_scatter(hist_v, [idx16], ones16)

    plsc.subcore_barrier()
    base = c * N_PAD + sid * ROWS_PER_TILE
    pltpu.sync_copy(s_sh.at[pl.ds(sid * ROWS_PER_TILE, ROWS_PER_TILE)],
                    s_out.at[pl.ds(base, ROWS_PER_TILE)])
    pltpu.sync_copy(hist_v, hist_out.at[pl.ds(wid * N_PAD, N_PAD)])


def _k1(x_flat, idx_flat):
    num_groups = idx_flat.shape[0] // GROUP_S
    return pl.kernel(
        functools.partial(_k1_body, num_groups),
        out_type=(
            jax.ShapeDtypeStruct((NC * N_PAD, 128), jnp.float32),
            jax.ShapeDtypeStruct((NW * N_PAD,), jnp.float32),
        ),
        mesh=_sc_mesh(),
        compiler_params=pltpu.CompilerParams(needs_layout_passes=False),
        scratch_types=[
            pltpu.VMEM_SHARED((N_PAD, 128), jnp.float32),
            pltpu.VMEM((2, GROUP_S), jnp.int32),
            pltpu.VMEM((2, GROUP_S, 128), jnp.float32),
            pltpu.VMEM((N_PAD,), jnp.float32),
            pltpu.SemaphoreType.DMA((2,)),
            pltpu.SemaphoreType.DMA((2,)),
        ],
    )(x_flat, idx_flat)


def _k2_kernel(s_ref, hist_ref, w_ref, b_ref, zn_ref, rdeg16_ref):
    s = s_ref[0:N_PAD, :] + s_ref[N_PAD:, :]
    deg = lax.dot_general(
        hist_ref[...], jnp.ones((NW, 1), jnp.float32),
        dimension_numbers=(((0,), (0,)), ((), ())),
        preferred_element_type=jnp.float32)
    rdeg = 1.0 / jnp.maximum(deg, 1.0)
    z = jnp.dot(s, w_ref[...], preferred_element_type=jnp.float32)
    z = z + deg * b_ref[...]
    zn_ref[...] = z * rdeg
    rdeg16_ref[...] = jnp.broadcast_to(rdeg, (N_PAD, 16))


def _k2(s_parts, hist2d, w_off, b_off2d):
    return pl.pallas_call(
        _k2_kernel,
        out_shape=(
            jax.ShapeDtypeStruct((N_PAD, 128), jnp.float32),
            jax.ShapeDtypeStruct((N_PAD, 16), jnp.float32),
        ),
    )(s_parts, hist2d, w_off, b_off2d)


def _k3_body(num_groups, g_base, zn_hbm, rdeg_hbm, idx_hbm, zg_out, rg_out,
             zn_sh, idx_v, zrows_v, rr_v, rdeg_v, sem_i, sem_g, sem_w, sem_wr):
    c = lax.axis_index("c")
    sid = lax.axis_index("s")
    wid = sid * NC + c

    pltpu.sync_copy(zn_hbm.at[pl.ds(sid * ROWS_PER_TILE, ROWS_PER_TILE)],
                    zn_sh.at[pl.ds(sid * ROWS_PER_TILE, ROWS_PER_TILE)])
    pltpu.sync_copy(rdeg_hbm, rdeg_v)
    plsc.subcore_barrier()
    lanes = lax.iota(jnp.int32, 16)
    zeros16 = jnp.zeros((16,), jnp.int32)
    n_i = (num_groups - wid + NW - 1) // NW

    def start_idx(i, slot):
        g = g_base + wid + i * NW
        pltpu.async_copy(idx_hbm.at[pl.ds(g * GROUP, GROUP)],
                         idx_v.at[slot], sem_i.at[slot])

    start_idx(0, 0)

    @pl.loop(0, n_i)
    def _(i):
        b = lax.rem(i, 2)
        g = g_base + wid + i * NW
        pltpu.make_async_copy(idx_hbm.at[pl.ds(g * GROUP, GROUP)],
                              idx_v.at[b], sem_i.at[b]).wait()

        @pl.when(i + 1 < n_i)
        def _():
            start_idx(i + 1, 1 - b)

        @pl.when(i >= 2)
        def _():
            g0 = pl.ds((g - g_base - 2 * NW) * GROUP, GROUP)
            g0r = pl.ds((g - g_base - 2 * NW) * GROUP * 16, GROUP * 16)
            pltpu.make_async_copy(zrows_v.at[b], zg_out.at[g0],
                                  sem_w.at[b]).wait()
            pltpu.make_async_copy(rr_v.at[b], rg_out.at[g0r],
                                  sem_wr.at[b]).wait()

        gather = pltpu.async_copy(zn_sh.at[idx_v.at[b]], zrows_v.at[b],
                                  sem_g.at[b])
        bvec = jnp.broadcast_to(b, (16,))
        for j in range(GROUP // 16):
            idx16 = idx_v[b, pl.ds(j * 16, 16)]
            r16 = plsc.load_gather(rdeg_v, [idx16])
            plsc.store_scatter(rr_v, [bvec, (lanes + (j * 16)) * 16], r16)
        gather.wait()

        gs = pl.ds((g - g_base) * GROUP, GROUP)
        pltpu.async_copy(zrows_v.at[b], zg_out.at[gs], sem_w.at[b])
        pltpu.async_copy(rr_v.at[b],
                         rg_out.at[pl.ds((g - g_base) * GROUP * 16, GROUP * 16)],
                         sem_wr.at[b])

    @pl.loop(0, 2)
    def _(s):
        pltpu.make_async_copy(zrows_v.at[s], zg_out.at[pl.ds(0, GROUP)],
                              sem_w.at[s]).wait()
        pltpu.make_async_copy(rr_v.at[s], rg_out.at[pl.ds(0, GROUP * 16)],
                              sem_wr.at[s]).wait()


def _k3(zn, rdeg, idx_flat, chunk, nchunks):
    num_groups = idx_flat.shape[0] // GROUP // nchunks
    ef = num_groups * GROUP
    return pl.kernel(
        functools.partial(_k3_body, num_groups, chunk * num_groups),
        out_type=(
            jax.ShapeDtypeStruct((ef, 128), jnp.float32),
            jax.ShapeDtypeStruct((ef * 16,), jnp.float32),
        ),
        mesh=_sc_mesh(),
        compiler_params=pltpu.CompilerParams(needs_layout_passes=False),
        scratch_types=[
            pltpu.VMEM_SHARED((N_PAD, 128), jnp.float32),
            pltpu.VMEM((2, GROUP), jnp.int32),
            pltpu.VMEM((2, GROUP, 128), jnp.float32),
            pltpu.VMEM((2, GROUP * 16), jnp.float32),
            pltpu.VMEM((N_PAD,), jnp.float32),
            pltpu.SemaphoreType.DMA((2,)),
            pltpu.SemaphoreType.DMA((2,)),
            pltpu.SemaphoreType.DMA((2,)),
            pltpu.SemaphoreType.DMA((2,)),
        ],
    )(zn, rdeg, idx_flat)


def _k4_kernel(x_ref, w_ref, b_ref, rg_ref, zg_ref, out_ref):
    y = jnp.dot(x_ref[...], w_ref[...], preferred_element_type=jnp.float32)
    y = y + b_ref[...]
    out_ref[...] = y * rg_ref[:, 0:1] + zg_ref[...]


def _k4(prev_out, x_flat, w_on, b_on2d, rg, zg, block, chunk, nchunks):
    ef = x_flat.shape[0]
    grid = ef // block // nchunks
    off = chunk * grid
    specs = [
        pl.BlockSpec((block, 128), lambda i: (i + off, 0)),
        pl.BlockSpec((128, 128), lambda i: (0, 0)),
        pl.BlockSpec((1, 128), lambda i: (0, 0)),
        pl.BlockSpec((block, 16), lambda i: (i, 0)),
        pl.BlockSpec((block, 128), lambda i: (i, 0)),
    ]
    out_shape = jax.ShapeDtypeStruct((ef, 128), jnp.float32)
    out_spec = pl.BlockSpec((block, 128), lambda i: (i + off, 0))
    if prev_out is None:
        return pl.pallas_call(
            _k4_kernel, grid=(grid,), in_specs=specs,
            out_specs=out_spec, out_shape=out_shape,
        )(x_flat, w_on, b_on2d, rg, zg)
    def body(o_ref, x_ref, w_ref, b_ref, rg_ref, zg_ref, out_ref):
        _k4_kernel(x_ref, w_ref, b_ref, rg_ref, zg_ref, out_ref)
    return pl.pallas_call(
        body, grid=(grid,),
        in_specs=[pl.BlockSpec(memory_space=pltpu.MemorySpace.HBM)] + specs,
        out_specs=out_spec, out_shape=out_shape,
        input_output_aliases={0: 0},
    )(prev_out, x_flat, w_on, b_on2d, rg, zg)


def kernel(x, edge_index, W_on, b_on, W_off, b_off):
    two, e, d_in = x.shape
    ef = two * e
    assert ef % GROUP == 0 and d_in == 128

    x_flat = x.reshape(ef, d_in)
    idx_flat = edge_index.reshape(ef)

    s_parts, hist = _k1(x_flat, idx_flat)
    zn, rdeg16 = _k2(s_parts, hist.reshape(NW, N_PAD), W_off,
                     b_off.reshape(1, 128))
    nchunks = 8
    out = None
    for k in range(nchunks):
        zg, rg1d = _k3(zn, rdeg16[:, 0], idx_flat, k, nchunks)
        rg = rg1d.reshape(-1, 16)
        out = _k4(out, x_flat, W_on, b_on.reshape(1, 128), rg, zg,
                  block=2000, chunk=k, nchunks=nchunks)
    return out.reshape(two, e, W_on.shape[1])


# speedup vs baseline: 1.1081x; 1.1081x over previous
import functools

import jax
import jax.numpy as jnp
from jax import lax
from jax.experimental import pallas as pl
from jax.experimental.pallas import tpu as pltpu, tpu_sc as plsc

N_NODES = 10000
N_PAD = 10240
GROUP = 128
GROUP_S = 128
NC, NS = 2, 16
NW = NC * NS
ROWS_PER_TILE = N_PAD // NS


def _sc_mesh():
    return plsc.VectorSubcoreMesh(
        core_axis_name="c", subcore_axis_name="s", num_cores=NC, num_subcores=NS
    )


def _k1_body(num_groups, x_hbm, idx_hbm, s_out, hist_out,
             s_sh, idx_v, rows_v, hist_v, sem_i, sem_r):
    c = lax.axis_index("c")
    sid = lax.axis_index("s")
    wid = sid * NC + c

    @pl.loop(0, GROUP_S)
    def _(i):
        @pl.loop(0, 8)
        def _(j):
            rows_v[0, i, pl.ds(j * 16, 16)] = jnp.zeros((16,), jnp.float32)

    @pl.loop(0, ROWS_PER_TILE // GROUP_S)
    def _(r):
        base = sid * ROWS_PER_TILE + r * GROUP_S
        pltpu.sync_copy(rows_v.at[0], s_sh.at[pl.ds(base, GROUP_S)])

    @pl.loop(0, N_PAD // 16)
    def _(i):
        hist_v[pl.ds(i * 16, 16)] = jnp.zeros((16,), jnp.float32)
    plsc.subcore_barrier()

    ones16 = jnp.ones((16,), jnp.float32)
    n_i = (num_groups - wid + NW - 1) // NW

    def start(i, slot):
        g = wid + i * NW
        pltpu.async_copy(idx_hbm.at[pl.ds(g * GROUP_S, GROUP_S)],
                         idx_v.at[slot], sem_i.at[slot])
        pltpu.async_copy(x_hbm.at[pl.ds(g * GROUP_S, GROUP_S)],
                         rows_v.at[slot], sem_r.at[slot])

    start(0, 0)

    @pl.loop(0, n_i)
    def _(i):
        b = lax.rem(i, 2)
        g = wid + i * NW
        pltpu.make_async_copy(idx_hbm.at[pl.ds(g * GROUP_S, GROUP_S)],
                              idx_v.at[b], sem_i.at[b]).wait()
        pltpu.make_async_copy(x_hbm.at[pl.ds(g * GROUP_S, GROUP_S)],
                              rows_v.at[b], sem_r.at[b]).wait()

        @pl.when(i + 1 < n_i)
        def _():
            start(i + 1, 1 - b)

        pltpu.sync_copy(rows_v.at[b], s_sh.at[idx_v.at[b]], add=True)
        for j in range(GROUP_S // 16):
            idx16 = idx_v[b, pl.ds(j * 16, 16)]
            plsc.addupdate_scatter(hist_v, [idx16], ones16)

    plsc.subcore_barrier()
    base = c * N_PAD + sid * ROWS_PER_TILE
    pltpu.sync_copy(s_sh.at[pl.ds(sid * ROWS_PER_TILE, ROWS_PER_TILE)],
                    s_out.at[pl.ds(base, ROWS_PER_TILE)])
    pltpu.sync_copy(hist_v, hist_out.at[pl.ds(wid * N_PAD, N_PAD)])


def _k1(x_flat, idx_flat):
    num_groups = idx_flat.shape[0] // GROUP_S
    return pl.kernel(
        functools.partial(_k1_body, num_groups),
        out_type=(
            jax.ShapeDtypeStruct((NC * N_PAD, 128), jnp.float32),
            jax.ShapeDtypeStruct((NW * N_PAD,), jnp.float32),
        ),
        mesh=_sc_mesh(),
        compiler_params=pltpu.CompilerParams(needs_layout_passes=False),
        scratch_types=[
            pltpu.VMEM_SHARED((N_PAD, 128), jnp.float32),
            pltpu.VMEM((2, GROUP_S), jnp.int32),
            pltpu.VMEM((2, GROUP_S, 128), jnp.float32),
            pltpu.VMEM((N_PAD,), jnp.float32),
            pltpu.SemaphoreType.DMA((2,)),
            pltpu.SemaphoreType.DMA((2,)),
        ],
    )(x_flat, idx_flat)


def _k2_kernel(s_ref, hist_ref, w_ref, b_ref, zn_ref, rdeg16_ref):
    s = s_ref[0:N_PAD, :] + s_ref[N_PAD:, :]
    deg = lax.dot_general(
        hist_ref[...], jnp.ones((NW, 1), jnp.float32),
        dimension_numbers=(((0,), (0,)), ((), ())),
        preferred_element_type=jnp.float32)
    rdeg = 1.0 / jnp.maximum(deg, 1.0)
    z = jnp.dot(s, w_ref[...], preferred_element_type=jnp.float32)
    z = z + deg * b_ref[...]
    zn_ref[...] = z * rdeg
    rdeg16_ref[...] = jnp.broadcast_to(rdeg, (N_PAD, 16))


def _k2(s_parts, hist2d, w_off, b_off2d):
    return pl.pallas_call(
        _k2_kernel,
        out_shape=(
            jax.ShapeDtypeStruct((N_PAD, 128), jnp.float32),
            jax.ShapeDtypeStruct((N_PAD, 16), jnp.float32),
        ),
    )(s_parts, hist2d, w_off, b_off2d)


def _k3_body(num_groups, g_base, zn_hbm, rdeg_hbm, idx_hbm, zg_out, rg_out,
             zn_sh, idx_v, zrows_v, rr_v, rdeg_v, sem_i, sem_g, sem_w, sem_wr):
    c = lax.axis_index("c")
    sid = lax.axis_index("s")
    wid = sid * NC + c

    pltpu.sync_copy(zn_hbm.at[pl.ds(sid * ROWS_PER_TILE, ROWS_PER_TILE)],
                    zn_sh.at[pl.ds(sid * ROWS_PER_TILE, ROWS_PER_TILE)])
    pltpu.sync_copy(rdeg_hbm, rdeg_v)
    plsc.subcore_barrier()
    lanes = lax.iota(jnp.int32, 16)
    zeros16 = jnp.zeros((16,), jnp.int32)
    n_i = (num_groups - wid + NW - 1) // NW

    def start_idx(i, slot):
        g = g_base + wid + i * NW
        pltpu.async_copy(idx_hbm.at[pl.ds(g * GROUP, GROUP)],
                         idx_v.at[slot], sem_i.at[slot])

    start_idx(0, 0)

    @pl.loop(0, n_i)
    def _(i):
        b = lax.rem(i, 2)
        g = g_base + wid + i * NW
        pltpu.make_async_copy(idx_hbm.at[pl.ds(g * GROUP, GROUP)],
                              idx_v.at[b], sem_i.at[b]).wait()

        @pl.when(i + 1 < n_i)
        def _():
            start_idx(i + 1, 1 - b)

        @pl.when(i >= 2)
        def _():
            g0 = pl.ds((g - g_base - 2 * NW) * GROUP, GROUP)
            g0r = pl.ds((g - g_base - 2 * NW) * GROUP * 16, GROUP * 16)
            pltpu.make_async_copy(zrows_v.at[b], zg_out.at[g0],
                                  sem_w.at[b]).wait()
            pltpu.make_async_copy(rr_v.at[b], rg_out.at[g0r],
                                  sem_wr.at[b]).wait()

        gather = pltpu.async_copy(zn_sh.at[idx_v.at[b]], zrows_v.at[b],
                                  sem_g.at[b])
        bvec = jnp.broadcast_to(b, (16,))
        for j in range(GROUP // 16):
            idx16 = idx_v[b, pl.ds(j * 16, 16)]
            r16 = plsc.load_gather(rdeg_v, [idx16])
            plsc.store_scatter(rr_v, [bvec, (lanes + (j * 16)) * 16], r16)
        gather.wait()

        gs = pl.ds((g - g_base) * GROUP, GROUP)
        pltpu.async_copy(zrows_v.at[b], zg_out.at[gs], sem_w.at[b])
        pltpu.async_copy(rr_v.at[b],
                         rg_out.at[pl.ds((g - g_base) * GROUP * 16, GROUP * 16)],
                         sem_wr.at[b])

    @pl.loop(0, 2)
    def _(s):
        pltpu.make_async_copy(zrows_v.at[s], zg_out.at[pl.ds(0, GROUP)],
                              sem_w.at[s]).wait()
        pltpu.make_async_copy(rr_v.at[s], rg_out.at[pl.ds(0, GROUP * 16)],
                              sem_wr.at[s]).wait()


def _k3(zn, rdeg, idx_flat, chunk, nchunks):
    num_groups = idx_flat.shape[0] // GROUP // nchunks
    ef = num_groups * GROUP
    return pl.kernel(
        functools.partial(_k3_body, num_groups, chunk * num_groups),
        out_type=(
            jax.ShapeDtypeStruct((ef, 128), jnp.float32),
            jax.ShapeDtypeStruct((ef * 16,), jnp.float32),
        ),
        mesh=_sc_mesh(),
        compiler_params=pltpu.CompilerParams(needs_layout_passes=False),
        scratch_types=[
            pltpu.VMEM_SHARED((N_PAD, 128), jnp.float32),
            pltpu.VMEM((2, GROUP), jnp.int32),
            pltpu.VMEM((2, GROUP, 128), jnp.float32),
            pltpu.VMEM((2, GROUP * 16), jnp.float32),
            pltpu.VMEM((N_PAD,), jnp.float32),
            pltpu.SemaphoreType.DMA((2,)),
            pltpu.SemaphoreType.DMA((2,)),
            pltpu.SemaphoreType.DMA((2,)),
            pltpu.SemaphoreType.DMA((2,)),
        ],
    )(zn, rdeg, idx_flat)


def _k4_kernel(x_ref, w_ref, b_ref, rg_ref, zg_ref, out_ref):
    y = jnp.dot(x_ref[...], w_ref[...], preferred_element_type=jnp.float32)
    y = y + b_ref[...]
    out_ref[...] = y * rg_ref[:, 0:1] + zg_ref[...]


def _k4(prev_out, x_flat, w_on, b_on2d, rg, zg, block, chunk, nchunks):
    ef = x_flat.shape[0]
    grid = ef // block // nchunks
    off = chunk * grid
    specs = [
        pl.BlockSpec((block, 128), lambda i: (i + off, 0)),
        pl.BlockSpec((128, 128), lambda i: (0, 0)),
        pl.BlockSpec((1, 128), lambda i: (0, 0)),
        pl.BlockSpec((block, 16), lambda i: (i, 0)),
        pl.BlockSpec((block, 128), lambda i: (i, 0)),
    ]
    out_shape = jax.ShapeDtypeStruct((ef, 128), jnp.float32)
    out_spec = pl.BlockSpec((block, 128), lambda i: (i + off, 0))
    if prev_out is None:
        return pl.pallas_call(
            _k4_kernel, grid=(grid,), in_specs=specs,
            out_specs=out_spec, out_shape=out_shape,
        )(x_flat, w_on, b_on2d, rg, zg)
    def body(o_ref, x_ref, w_ref, b_ref, rg_ref, zg_ref, out_ref):
        _k4_kernel(x_ref, w_ref, b_ref, rg_ref, zg_ref, out_ref)
    return pl.pallas_call(
        body, grid=(grid,),
        in_specs=[pl.BlockSpec(memory_space=pltpu.MemorySpace.HBM)] + specs,
        out_specs=out_spec, out_shape=out_shape,
        input_output_aliases={0: 0},
    )(prev_out, x_flat, w_on, b_on2d, rg, zg)


def kernel(x, edge_index, W_on, b_on, W_off, b_off):
    two, e, d_in = x.shape
    ef = two * e
    assert ef % GROUP == 0 and d_in == 128

    x_flat = x.reshape(ef, d_in)
    idx_flat = edge_index.reshape(ef)

    s_parts, hist = _k1(x_flat, idx_flat)
    zn, rdeg16 = _k2(s_parts, hist.reshape(NW, N_PAD), W_off,
                     b_off.reshape(1, 128))
    nchunks = 4
    out = None
    for k in range(nchunks):
        zg, rg1d = _k3(zn, rdeg16[:, 0], idx_flat, k, nchunks)
        rg = rg1d.reshape(-1, 16)
        out = _k4(out, x_flat, W_on, b_on.reshape(1, 128), rg, zg,
                  block=6400, chunk=k, nchunks=nchunks)
    return out.reshape(two, e, W_on.shape[1])
